# Initial kernel scaffold; baseline (speedup 1.0000x reference)
import jax
import jax.numpy as jnp
from jax.experimental import pallas as pl


def kernel(x, edge_index, Wl1, bl1, Wr1, Wl2, bl2, Wr2, Wc, bc):
    raise NotImplementedError("write your pallas kernel here")


# speedup vs baseline: 8.3418x; 8.3418x over previous
import functools

import jax
import jax.numpy as jnp
from jax import lax
from jax.experimental import pallas as pl
from jax.experimental.pallas import tpu as pltpu
from jax.experimental.pallas import tpu_sc as plsc

N_NODES = 10000
N_EDGES = 320000
IN_CH = 128
HID = 64
HID2 = 32
OUT_CH = 1

NUM_SC = 2
NUM_TILES = 16
NW = NUM_SC * NUM_TILES
CHUNK = 128
CHUNKS_PER_TILE = 79
EDGES_PER_TILE = CHUNK * CHUNKS_PER_TILE
E_PAD = EDGES_PER_TILE * NW
DUMMY_ROW = N_NODES
ACC_ROWS = 10112
STRIPE = ACC_ROWS // NUM_TILES
LAST_STRIPE = N_NODES - (NUM_TILES - 1) * STRIPE

_MESH = plsc.VectorSubcoreMesh(core_axis_name="c", subcore_axis_name="s")
_SC_PARAMS = pltpu.CompilerParams(use_tc_tiling_on_sc=False)
_MM = (((1,), (0,)), ((), ()))


def _zero_fill(buf, nrows, width):
    @pl.loop(0, nrows)
    def _(i):
        @pl.loop(0, width // 16)
        def _(k):
            buf[i, pl.ds(k * 16, 16)] = jnp.zeros((16,), jnp.float32)


def _make_sc_agg(width):
    out_t = [jax.ShapeDtypeStruct((N_NODES, width), jnp.float32)] * 2

    @functools.partial(
        pl.kernel,
        out_type=out_t,
        mesh=_MESH,
        compiler_params=_SC_PARAMS,
        scratch_types=[
            pltpu.VMEM((CHUNKS_PER_TILE, CHUNK), jnp.int32),
            pltpu.VMEM((CHUNKS_PER_TILE, CHUNK), jnp.int32),
            pltpu.VMEM((CHUNK, width), jnp.float32),
            pltpu.VMEM_SHARED((ACC_ROWS, width), jnp.float32),
        ],
    )
    def agg(table_hbm, src_hbm, dst_hbm, out0, out1, src_v, dst_v, rows_v, acc):
        c = lax.axis_index("c")
        s = lax.axis_index("s")
        w = c * NUM_TILES + s
        pltpu.sync_copy(src_hbm.at[w], src_v)
        pltpu.sync_copy(dst_hbm.at[w], dst_v)
        _zero_fill(rows_v, CHUNK, width)
        zbase = s * STRIPE
        for off in range(0, STRIPE, CHUNK):
            nrow = min(CHUNK, STRIPE - off)
            pltpu.sync_copy(rows_v.at[pl.ds(0, nrow)],
                            acc.at[pl.ds(zbase + off, nrow)])
        plsc.subcore_barrier()

        @pl.loop(0, CHUNKS_PER_TILE)
        def _(j):
            pltpu.sync_copy(table_hbm.at[src_v.at[j]], rows_v)
            pltpu.sync_copy(rows_v, acc.at[dst_v.at[j]], add=True)

        plsc.subcore_barrier()
        _copy_out(acc, out0, out1, c, s)

    return agg


def _copy_out(acc, out0, out1, c, s):
    ob = s * STRIPE

    def stripe_to(dst):
        @pl.when(s < NUM_TILES - 1)
        def _():
            pltpu.sync_copy(acc.at[pl.ds(ob, STRIPE)],
                            dst.at[pl.ds(ob, STRIPE)])

        @pl.when(s == NUM_TILES - 1)
        def _():
            pltpu.sync_copy(acc.at[pl.ds(ob, LAST_STRIPE)],
                            dst.at[pl.ds(ob, LAST_STRIPE)])

    @pl.when(c == 0)
    def _():
        stripe_to(out0)

    @pl.when(c == 1)
    def _():
        stripe_to(out1)


_CNT_W = 16


def _sc_count(dst_hbm_arr):
    out_t = [jax.ShapeDtypeStruct((N_NODES, _CNT_W), jnp.float32)] * 2

    @functools.partial(
        pl.kernel,
        out_type=out_t,
        mesh=_MESH,
        compiler_params=_SC_PARAMS,
        scratch_types=[
            pltpu.VMEM((CHUNKS_PER_TILE, CHUNK), jnp.int32),
            pltpu.VMEM((CHUNK, _CNT_W), jnp.float32),
            pltpu.VMEM_SHARED((ACC_ROWS, _CNT_W), jnp.float32),
        ],
    )
    def cnt(dst_hbm, out0, out1, dst_v, ones_v, acc):
        c = lax.axis_index("c")
        s = lax.axis_index("s")
        w = c * NUM_TILES + s
        pltpu.sync_copy(dst_hbm.at[w], dst_v)
        _zero_fill(ones_v, CHUNK, _CNT_W)
        zbase = s * STRIPE
        for off in range(0, STRIPE, CHUNK):
            nrow = min(CHUNK, STRIPE - off)
            pltpu.sync_copy(ones_v.at[pl.ds(0, nrow)],
                            acc.at[pl.ds(zbase + off, nrow)])

        @pl.loop(0, CHUNK)
        def _(i):
            ones_v[i, pl.ds(0, _CNT_W)] = jnp.ones((_CNT_W,), jnp.float32)

        plsc.subcore_barrier()

        @pl.loop(0, CHUNKS_PER_TILE)
        def _(j):
            pltpu.sync_copy(ones_v, acc.at[dst_v.at[j]], add=True)

        plsc.subcore_barrier()
        _copy_out(acc, out0, out1, c, s)

    return cnt(dst_hbm_arr)


_BLK = 2000


def _tc_pre(x, wl, wr, b):
    def body(x_ref, wl_ref, wr_ref, b_ref, xl_ref, xr_ref):
        xb = x_ref[...]
        xl_ref[...] = lax.dot_general(xb, wl_ref[...], _MM,
                                      precision=lax.Precision.HIGHEST)
        xr_ref[...] = lax.dot_general(xb, wr_ref[...], _MM,
                                      precision=lax.Precision.HIGHEST) + b_ref[...]

    return pl.pallas_call(
        body,
        grid=(N_NODES // _BLK,),
        in_specs=[
            pl.BlockSpec((_BLK, IN_CH), lambda i: (i, 0)),
            pl.BlockSpec((IN_CH, HID), lambda i: (0, 0)),
            pl.BlockSpec((IN_CH, HID), lambda i: (0, 0)),
            pl.BlockSpec((1, HID), lambda i: (0, 0)),
        ],
        out_specs=[pl.BlockSpec((_BLK, HID), lambda i: (i, 0))] * 2,
        out_shape=[jax.ShapeDtypeStruct((N_NODES, HID), jnp.float32)] * 2,
    )(x, wl, wr, b)


def _tc_mid(a0, a1, xr, c0, c1, wl, wr, b):
    def body(a0_ref, a1_ref, xr_ref, c0_ref, c1_ref, wl_ref, wr_ref, b_ref,
             hl_ref, hr_ref):
        cntv = jnp.maximum(c0_ref[:, 0:1] + c1_ref[:, 0:1], 1.0)
        h1 = jnp.maximum((a0_ref[...] + a1_ref[...]) / cntv + xr_ref[...], 0.0)
        hl_ref[...] = lax.dot_general(h1, wl_ref[...], _MM,
                                      precision=lax.Precision.HIGHEST)
        hr_ref[...] = lax.dot_general(h1, wr_ref[...], _MM,
                                      precision=lax.Precision.HIGHEST) + b_ref[...]

    return pl.pallas_call(
        body,
        grid=(N_NODES // _BLK,),
        in_specs=[
            pl.BlockSpec((_BLK, HID), lambda i: (i, 0)),
            pl.BlockSpec((_BLK, HID), lambda i: (i, 0)),
            pl.BlockSpec((_BLK, HID), lambda i: (i, 0)),
            pl.BlockSpec((_BLK, _CNT_W), lambda i: (i, 0)),
            pl.BlockSpec((_BLK, _CNT_W), lambda i: (i, 0)),
            pl.BlockSpec((HID, HID2), lambda i: (0, 0)),
            pl.BlockSpec((HID, HID2), lambda i: (0, 0)),
            pl.BlockSpec((1, HID2), lambda i: (0, 0)),
        ],
        out_specs=[pl.BlockSpec((_BLK, HID2), lambda i: (i, 0))] * 2,
        out_shape=[jax.ShapeDtypeStruct((N_NODES, HID2), jnp.float32)] * 2,
    )(a0, a1, xr, c0, c1, wl, wr, b)


def _tc_post(a0, a1, hr, c0, c1, wc, bc):
    def body(a0_ref, a1_ref, hr_ref, c0_ref, c1_ref, wc_ref, bc_ref, o_ref):
        cntv = jnp.maximum(c0_ref[:, 0:1] + c1_ref[:, 0:1], 1.0)
        h2 = jnp.maximum((a0_ref[...] + a1_ref[...]) / cntv + hr_ref[...], 0.0)
        o_ref[...] = lax.dot_general(h2, wc_ref[...], _MM,
                                     precision=lax.Precision.HIGHEST) + bc_ref[...]

    return pl.pallas_call(
        body,
        grid=(N_NODES // _BLK,),
        in_specs=[
            pl.BlockSpec((_BLK, HID2), lambda i: (i, 0)),
            pl.BlockSpec((_BLK, HID2), lambda i: (i, 0)),
            pl.BlockSpec((_BLK, HID2), lambda i: (i, 0)),
            pl.BlockSpec((_BLK, _CNT_W), lambda i: (i, 0)),
            pl.BlockSpec((_BLK, _CNT_W), lambda i: (i, 0)),
            pl.BlockSpec((HID2, OUT_CH), lambda i: (0, 0)),
            pl.BlockSpec((1, OUT_CH), lambda i: (0, 0)),
        ],
        out_specs=pl.BlockSpec((_BLK, OUT_CH), lambda i: (i, 0)),
        out_shape=jax.ShapeDtypeStruct((N_NODES, OUT_CH), jnp.float32),
    )(a0, a1, hr, c0, c1, wc, bc)


def kernel(x, edge_index, Wl1, bl1, Wr1, Wl2, bl2, Wr2, Wc, bc):
    ei = edge_index.astype(jnp.int32)
    pad = E_PAD - N_EDGES
    src = jnp.concatenate([ei[0], jnp.zeros((pad,), jnp.int32)])
    dst = jnp.concatenate([ei[1], jnp.full((pad,), DUMMY_ROW, jnp.int32)])
    src = src.reshape(NW, CHUNKS_PER_TILE, CHUNK)
    dst = dst.reshape(NW, CHUNKS_PER_TILE, CHUNK)

    cnt0, cnt1 = _sc_count(dst)
    xl1, xr1 = _tc_pre(x, Wl1.T, Wr1.T, bl1.reshape(1, HID))
    a0, a1 = _make_sc_agg(HID)(xl1, src, dst)
    hl2, hr2 = _tc_mid(a0, a1, xr1, cnt0, cnt1, Wl2.T, Wr2.T,
                       bl2.reshape(1, HID2))
    b0, b1 = _make_sc_agg(HID2)(hl2, src, dst)
    return _tc_post(b0, b1, hr2, cnt0, cnt1, Wc.T, bc.reshape(1, OUT_CH))
